# baseline (speedup 1.0000x reference)
import functools

import jax
import jax.numpy as jnp
import numpy as np
from jax.experimental import pallas as pl

N_RAYS = 4096
S = 524288
K0_DIM = 12
VIEWBASE_PE = 4
RGBNET_WIDTH = 128
ALPHA_INIT = 1e-6
_ACT_SHIFT = float(np.log(1.0 / (1.0 - ALPHA_INIT) - 1.0))

_E_EXPAND = np.zeros((3, 12), dtype=np.float32)
for _d in range(3):
    for _f in range(VIEWBASE_PE):
        _E_EXPAND[_d, _d * 4 + _f] = float(2.0 ** _f)
_E_EXPAND = jnp.asarray(_E_EXPAND)

_BLK = 2048


def _tc_body(dens_ref, k0_ref, vd_ref, e_ref, w0_ref, b0_ref, w1_ref, b1_ref,
             w2_ref, b2_ref, log1m_ref, argb_ref):
    x = dens_ref[:]
    e = jnp.exp(x + _ACT_SHIFT)
    t = jax.lax.rsqrt(1.0 + e)
    alpha = 1.0 - t
    log1m_ref[:] = jnp.log(jnp.clip(t, 1e-10, 1.0))

    vd = vd_ref[:]
    inv = jax.lax.rsqrt(jnp.sum(vd * vd, axis=-1, keepdims=True))
    norm = jnp.sqrt(jnp.sum(vd * vd, axis=-1, keepdims=True)) + 1e-8
    del inv
    vdn = vd / norm
    vf = jnp.dot(vdn, e_ref[:], preferred_element_type=jnp.float32)

    w0 = w0_ref[:]
    h0 = (jnp.dot(k0_ref[:], w0[0:12], preferred_element_type=jnp.float32)
          + jnp.dot(vdn, w0[12:15], preferred_element_type=jnp.float32)
          + jnp.dot(jnp.sin(vf), w0[15:27], preferred_element_type=jnp.float32)
          + jnp.dot(jnp.cos(vf), w0[27:39], preferred_element_type=jnp.float32)
          + b0_ref[:][None, :])
    h0 = jnp.maximum(h0, 0.0)
    h1 = jnp.maximum(jnp.dot(h0, w1_ref[:], preferred_element_type=jnp.float32)
                     + b1_ref[:][None, :], 0.0)
    logit = jnp.dot(h1, w2_ref[:], preferred_element_type=jnp.float32) + b2_ref[:][None, :]
    rgb = jax.nn.sigmoid(logit)
    argb_ref[:] = alpha[:, None] * rgb


@functools.partial(jax.jit, static_argnames=())
def _tc_stage(density, k0_feat, viewdirs, W0, b0, W1, b1, W2, b2):
    grid = (S // _BLK,)
    return pl.pallas_call(
        _tc_body,
        grid=grid,
        in_specs=[
            pl.BlockSpec((_BLK,), lambda i: (i,)),
            pl.BlockSpec((_BLK, K0_DIM), lambda i: (i, 0)),
            pl.BlockSpec((_BLK, 3), lambda i: (i, 0)),
            pl.BlockSpec((3, 12), lambda i: (0, 0)),
            pl.BlockSpec((39, RGBNET_WIDTH), lambda i: (0, 0)),
            pl.BlockSpec((RGBNET_WIDTH,), lambda i: (0,)),
            pl.BlockSpec((RGBNET_WIDTH, RGBNET_WIDTH), lambda i: (0, 0)),
            pl.BlockSpec((RGBNET_WIDTH,), lambda i: (0,)),
            pl.BlockSpec((RGBNET_WIDTH, 3), lambda i: (0, 0)),
            pl.BlockSpec((3,), lambda i: (0,)),
        ],
        out_specs=[
            pl.BlockSpec((_BLK,), lambda i: (i,)),
            pl.BlockSpec((_BLK, 3), lambda i: (i, 0)),
        ],
        out_shape=[
            jax.ShapeDtypeStruct((S,), jnp.float32),
            jax.ShapeDtypeStruct((S, 3), jnp.float32),
        ],
    )(density, k0_feat, viewdirs, _E_EXPAND, W0, b0, W1, b1, W2, b2)


def kernel(density, k0_feat, viewdirs, ray_id, W0, b0, W1, b1, W2, b2):
    log1m, argb = _tc_stage(density, k0_feat, viewdirs, W0, b0, W1, b1, W2, b2)
    csum = jnp.cumsum(log1m)
    excl = csum - log1m
    offset = jax.ops.segment_max(excl, ray_id, num_segments=N_RAYS)
    off_g = offset[ray_id]
    T = jnp.exp(excl - off_g)
    lastv = jax.ops.segment_min(csum - off_g, ray_id, num_segments=N_RAYS)
    alphainv_last = jnp.exp(jnp.where(jnp.isfinite(lastv), lastv, 0.0))
    rgb_marched = jax.ops.segment_sum(T[:, None] * argb, ray_id, num_segments=N_RAYS)
    return rgb_marched + alphainv_last[:, None]


# speedup vs baseline: 4.3518x; 4.3518x over previous
import functools

import jax
import jax.numpy as jnp
import numpy as np
from jax import lax
from jax.experimental import pallas as pl
from jax.experimental.pallas import tpu as pltpu
from jax.experimental.pallas import tpu_sc as plsc

N_RAYS = 4096
S = 524288
K0_DIM = 12
VIEWBASE_PE = 4
RGBNET_WIDTH = 128
ALPHA_INIT = 1e-6
_ACT_SHIFT = float(np.log(1.0 / (1.0 - ALPHA_INIT) - 1.0))

_E_EXPAND = np.zeros((3, 12), dtype=np.float32)
for _d in range(3):
    for _f in range(VIEWBASE_PE):
        _E_EXPAND[_d, _d * 4 + _f] = float(2.0 ** _f)

_BLK = 2048


def _tc_body(dens_ref, k0_ref, vd_ref, e_ref, w0_ref, b0_ref, w1_ref, b1_ref,
             w2_ref, b2_ref, log1m_ref, argb_ref):
    x = dens_ref[:]
    e = jnp.exp(x + _ACT_SHIFT)
    t = jax.lax.rsqrt(1.0 + e)
    alpha = 1.0 - t
    log1m_ref[:] = jnp.log(jnp.clip(t, 1e-10, 1.0))

    vd = vd_ref[:]
    inv = jax.lax.rsqrt(jnp.sum(vd * vd, axis=-1, keepdims=True))
    norm = jnp.sqrt(jnp.sum(vd * vd, axis=-1, keepdims=True)) + 1e-8
    del inv
    vdn = vd / norm
    vf = jnp.dot(vdn, e_ref[:], preferred_element_type=jnp.float32)

    w0 = w0_ref[:]
    h0 = (jnp.dot(k0_ref[:], w0[0:12], preferred_element_type=jnp.float32)
          + jnp.dot(vdn, w0[12:15], preferred_element_type=jnp.float32)
          + jnp.dot(jnp.sin(vf), w0[15:27], preferred_element_type=jnp.float32)
          + jnp.dot(jnp.cos(vf), w0[27:39], preferred_element_type=jnp.float32)
          + b0_ref[:][None, :])
    h0 = jnp.maximum(h0, 0.0)
    h1 = jnp.maximum(jnp.dot(h0, w1_ref[:], preferred_element_type=jnp.float32)
                     + b1_ref[:][None, :], 0.0)
    logit = jnp.dot(h1, w2_ref[:], preferred_element_type=jnp.float32) + b2_ref[:][None, :]
    rgb = jax.nn.sigmoid(logit)
    argb_ref[:] = alpha[:, None] * rgb


@functools.partial(jax.jit, static_argnames=())
def _tc_stage(density, k0_feat, viewdirs, W0, b0, W1, b1, W2, b2):
    grid = (S // _BLK,)
    return pl.pallas_call(
        _tc_body,
        grid=grid,
        in_specs=[
            pl.BlockSpec((_BLK,), lambda i: (i,)),
            pl.BlockSpec((_BLK, K0_DIM), lambda i: (i, 0)),
            pl.BlockSpec((_BLK, 3), lambda i: (i, 0)),
            pl.BlockSpec((3, 12), lambda i: (0, 0)),
            pl.BlockSpec((39, RGBNET_WIDTH), lambda i: (0, 0)),
            pl.BlockSpec((RGBNET_WIDTH,), lambda i: (0,)),
            pl.BlockSpec((RGBNET_WIDTH, RGBNET_WIDTH), lambda i: (0, 0)),
            pl.BlockSpec((RGBNET_WIDTH,), lambda i: (0,)),
            pl.BlockSpec((RGBNET_WIDTH, 3), lambda i: (0, 0)),
            pl.BlockSpec((3,), lambda i: (0,)),
        ],
        out_specs=[
            pl.BlockSpec((_BLK,), lambda i: (i,)),
            pl.BlockSpec((_BLK, 3), lambda i: (i, 0)),
        ],
        out_shape=[
            jax.ShapeDtypeStruct((S,), jnp.float32),
            jax.ShapeDtypeStruct((S, 3), jnp.float32),
        ],
    )(density, k0_feat, viewdirs, _E_EXPAND, W0, b0, W1, b1, W2, b2)


_NW = 16
_CHUNK = S // _NW
_NV = _CHUNK // 16
_SUB = 4096
_NSUB = _CHUNK // _SUB
_SUBV = _SUB // 16
_RPW = N_RAYS // _NW
_RG = _RPW // 16


def _sc_body(rid_hbm, lg_hbm, argb_hbm, out_hbm,
             rid_buf, ex_buf, argb_buf,
             off_loc, lastc_loc, acc_loc, off_tab,
             red, tot_v, fin_v, out_v, sc17, tot_stage,
             totals_sh, sh_big, fin_sh):
    cid = lax.axis_index("c")
    sid = lax.axis_index("s")

    @pl.when(cid == 0)
    def _core0():
        w = sid
        base = w * _CHUNK
        iota = lax.iota(jnp.int32, 16)
        zf = jnp.zeros((16,), jnp.float32)

        def _z(i, c):
            off_loc[pl.ds(16 * i, 16)] = zf
            lastc_loc[pl.ds(16 * i, 16)] = zf
            for ch in range(3):
                acc_loc[ch, pl.ds(16 * i, 16)] = zf
            return c
        lax.fori_loop(0, N_RAYS // 16, _z, 0)
        sc17[pl.ds(0, 16)] = jnp.full((16,), -1.0, jnp.float32)

        pltpu.sync_copy(rid_hbm.at[pl.ds(base, _CHUNK)],
                        rid_buf.at[pl.ds(16, _CHUNK)])

        @pl.when(w > 0)
        def _():
            pltpu.sync_copy(rid_hbm.at[pl.ds(base - 16, 16)],
                            rid_buf.at[pl.ds(0, 16)])

        @pl.when(w == 0)
        def _():
            rid_buf[pl.ds(0, 16)] = jnp.full((16,), -1, jnp.int32)

        @pl.when(w < _NW - 1)
        def _():
            pltpu.sync_copy(rid_hbm.at[pl.ds(base + _CHUNK, 16)],
                            rid_buf.at[pl.ds(16 + _CHUNK, 16)])

        @pl.when(w == _NW - 1)
        def _():
            rid_buf[pl.ds(16 + _CHUNK, 16)] = jnp.full((16,), -1, jnp.int32)

        def _p1sub(sub, carry):
            pltpu.sync_copy(lg_hbm.at[pl.ds(base + sub * _SUB, _SUB)],
                            ex_buf.at[pl.ds(sub * _SUB, _SUB)])

            def _p1v(j, c):
                o = sub * _SUB + 16 * j
                v = ex_buf[pl.ds(o, 16)]
                s = plsc.cumsum(v)
                ex_buf[pl.ds(o, 16)] = (c + s) - v
                return c + jnp.sum(v)
            return lax.fori_loop(0, _SUBV, _p1v, carry)
        total = lax.fori_loop(0, _NSUB, _p1sub, jnp.float32(0.0))
        ex_buf[pl.ds(_CHUNK, 16)] = zf + total

        tot_stage[pl.ds(0, 16)] = zf + total
        pltpu.sync_copy(tot_stage, totals_sh.at[w])
        plsc.subcore_barrier()
        pltpu.sync_copy(totals_sh, tot_v)
        tvals = plsc.load_gather(tot_v, [iota, jnp.zeros((16,), jnp.int32)])
        prefix = jnp.sum(jnp.where(iota < w, tvals, 0.0))

        def _p2(j, c):
            rv = rid_buf[pl.ds(16 + 16 * j, 16)]
            rp = plsc.load_gather(rid_buf, [15 + 16 * j + iota])
            rn = plsc.load_gather(rid_buf, [17 + 16 * j + iota])
            exg = ex_buf[pl.ds(16 * j, 16)] + prefix
            exn = plsc.load_gather(ex_buf, [16 * j + 1 + iota]) + prefix
            plsc.store_scatter(off_loc, [rv], exg, mask=rv != rp)
            plsc.store_scatter(lastc_loc, [rv], exn, mask=rv != rn)
            return c
        lax.fori_loop(0, _NV, _p2, 0)
        pltpu.sync_copy(off_loc, sh_big.at[w, 0])
        pltpu.sync_copy(lastc_loc, sh_big.at[w, 1])
        plsc.subcore_barrier()

        c0 = w * _RPW
        for t in range(2):
            for k in range(_NW):
                pltpu.sync_copy(sh_big.at[k, t, pl.ds(c0, _RPW)], red.at[k])
            for g in range(_RG):
                def _radd(k, a, g=g):
                    return a + red[k, pl.ds(16 * g, 16)]
                fin_v[t, pl.ds(16 * g, 16)] = lax.fori_loop(0, _NW, _radd, zf)
        pltpu.sync_copy(fin_v.at[0], fin_sh.at[0, pl.ds(c0, _RPW)])
        pltpu.sync_copy(fin_v.at[1], fin_sh.at[1, pl.ds(c0, _RPW)])
        plsc.subcore_barrier()
        pltpu.sync_copy(fin_sh.at[0], off_tab)

        def _p3sub(sub, carry):
            pltpu.sync_copy(
                argb_hbm.at[pl.ds((base + sub * _SUB) * 3, _SUB * 3)],
                argb_buf)

            def _p3v(j, cr):
                accs = [cr[0], cr[1], cr[2]]
                pbs = [cr[3], cr[4], cr[5]]
                jj = sub * _SUBV + j
                rv = rid_buf[pl.ds(16 + 16 * jj, 16)]
                rn = plsc.load_gather(rid_buf, [17 + 16 * jj + iota])
                exg = ex_buf[pl.ds(16 * jj, 16)] + prefix
                og = plsc.load_gather(off_tab, [rv])
                tw = jnp.exp(exg - og)
                lf = jnp.logical_or(
                    rv != rn,
                    jnp.logical_and(jj == _NV - 1, iota == 15))
                rows = 16 * j + iota
                out = []
                for ch in range(3):
                    a = plsc.load_gather(argb_buf, [3 * rows + ch])
                    v = tw * a
                    s = plsc.cumsum(v) + accs[ch]
                    wv = jnp.where(lf, s, -1.0)
                    plsc.store_scatter(sc17, [iota + 1], wv)
                    prevc = jnp.maximum(plsc.cummax(sc17[pl.ds(0, 16)]),
                                        pbs[ch])
                    plsc.store_scatter(
                        acc_loc, [jnp.full((16,), ch, jnp.int32), rv],
                        s - prevc, mask=lf)
                    out.append((accs[ch] + jnp.sum(v),
                                jnp.maximum(pbs[ch], jnp.max(wv))))
                return (out[0][0], out[1][0], out[2][0],
                        out[0][1], out[1][1], out[2][1])
            return lax.fori_loop(0, _SUBV, _p3v, carry)
        z6 = (jnp.float32(0.0),) * 6
        lax.fori_loop(0, _NSUB, _p3sub, z6)
        pltpu.sync_copy(acc_loc, sh_big.at[w])
        plsc.subcore_barrier()

        pltpu.sync_copy(fin_sh.at[0, pl.ds(c0, _RPW)], fin_v.at[0])
        pltpu.sync_copy(fin_sh.at[1, pl.ds(c0, _RPW)], fin_v.at[1])
        for g in range(_RG):
            offv = fin_v[0, pl.ds(16 * g, 16)]
            lcv = fin_v[1, pl.ds(16 * g, 16)]
            fin_v[0, pl.ds(16 * g, 16)] = jnp.exp(lcv - offv)
        for ch in range(3):
            for k in range(_NW):
                pltpu.sync_copy(sh_big.at[k, ch, pl.ds(c0, _RPW)], red.at[k])
            for g in range(_RG):
                def _r3(k, a, g=g):
                    return a + red[k, pl.ds(16 * g, 16)]
                av = lax.fori_loop(0, _NW, _r3, zf)
                rows = 16 * g + iota
                plsc.store_scatter(out_v, [3 * rows + ch],
                                   av + fin_v[0, pl.ds(16 * g, 16)])
        pltpu.sync_copy(out_v, out_hbm.at[pl.ds(c0 * 3, _RPW * 3)])


_sc_stage = pl.kernel(
    _sc_body,
    out_type=jax.ShapeDtypeStruct((N_RAYS * 3,), jnp.float32),
    mesh=plsc.VectorSubcoreMesh(core_axis_name="c", subcore_axis_name="s"),
    compiler_params=pltpu.CompilerParams(
        needs_layout_passes=False, use_tc_tiling_on_sc=False),
    scratch_types=[
        pltpu.VMEM((_CHUNK + 32,), jnp.int32),
        pltpu.VMEM((_CHUNK + 16,), jnp.float32),
        pltpu.VMEM((_SUB * 3,), jnp.float32),
        pltpu.VMEM((N_RAYS,), jnp.float32),
        pltpu.VMEM((N_RAYS,), jnp.float32),
        pltpu.VMEM((3, N_RAYS), jnp.float32),
        pltpu.VMEM((N_RAYS,), jnp.float32),
        pltpu.VMEM((_NW, _RPW), jnp.float32),
        pltpu.VMEM((16, 16), jnp.float32),
        pltpu.VMEM((2, _RPW), jnp.float32),
        pltpu.VMEM((_RPW * 3,), jnp.float32),
        pltpu.VMEM((32,), jnp.float32),
        pltpu.VMEM((16,), jnp.float32),
        pltpu.VMEM_SHARED((16, 16), jnp.float32),
        pltpu.VMEM_SHARED((_NW, 3, N_RAYS), jnp.float32),
        pltpu.VMEM_SHARED((2, N_RAYS), jnp.float32),
    ],
)


def kernel(density, k0_feat, viewdirs, ray_id, W0, b0, W1, b1, W2, b2):
    log1m, argb = _tc_stage(density, k0_feat, viewdirs, W0, b0, W1, b1, W2, b2)
    flat = _sc_stage(ray_id, log1m, argb.reshape(-1))
    return flat.reshape(N_RAYS, 3)


# speedup vs baseline: 11.0436x; 2.5377x over previous
import functools

import jax
import jax.numpy as jnp
import numpy as np
from jax import lax
from jax.experimental import pallas as pl
from jax.experimental.pallas import tpu as pltpu
from jax.experimental.pallas import tpu_sc as plsc

N_RAYS = 4096
S = 524288
K0_DIM = 12
VIEWBASE_PE = 4
RGBNET_WIDTH = 128
ALPHA_INIT = 1e-6
_ACT_SHIFT = float(np.log(1.0 / (1.0 - ALPHA_INIT) - 1.0))

_W0_PERM = np.arange(39)
for _b in (15, 27):
    for _f in range(VIEWBASE_PE):
        for _d in range(3):
            _W0_PERM[_b + 3 * _f + _d] = _b + _d * 4 + _f

_BLK = 2048


def _tc_body(dens_ref, k0t_ref, vdt_ref, w0t_ref, b0_ref, w1t_ref, b1_ref,
             w2t_ref, b2_ref, log1m_ref, argb_ref):
    x = dens_ref[:]
    e = jnp.exp(x + _ACT_SHIFT)
    t = jax.lax.rsqrt(1.0 + e)
    alpha = 1.0 - t
    log1m_ref[:] = jnp.log(jnp.clip(t, 1e-10, 1.0))

    vd = vdt_ref[:]
    norm = jnp.sqrt(jnp.sum(vd * vd, axis=0, keepdims=True)) + 1e-8
    vdn = vd / norm

    s = jnp.sin(vdn)
    c = jnp.cos(vdn)
    sc = [(s, c)]
    for _ in range(VIEWBASE_PE - 1):
        s, c = 2.0 * s * c, 1.0 - 2.0 * s * s
        sc.append((s, c))

    w0t = w0t_ref[:]
    h0 = (jax.lax.dot_general(w0t[:, 0:12], k0t_ref[:],
                              (((1,), (0,)), ((), ())),
                              preferred_element_type=jnp.float32)
          + jax.lax.dot_general(w0t[:, 12:15], vdn,
                                (((1,), (0,)), ((), ())),
                                preferred_element_type=jnp.float32)
          + b0_ref[:])
    for f in range(VIEWBASE_PE):
        h0 = h0 + jax.lax.dot_general(w0t[:, 15 + 3 * f:18 + 3 * f],
                                      sc[f][0], (((1,), (0,)), ((), ())),
                                      preferred_element_type=jnp.float32)
        h0 = h0 + jax.lax.dot_general(w0t[:, 27 + 3 * f:30 + 3 * f],
                                      sc[f][1], (((1,), (0,)), ((), ())),
                                      preferred_element_type=jnp.float32)
    h0 = jnp.maximum(h0, 0.0)
    h1 = jnp.maximum(
        jax.lax.dot_general(w1t_ref[:], h0, (((1,), (0,)), ((), ())),
                            preferred_element_type=jnp.float32) + b1_ref[:],
        0.0)
    logit = jax.lax.dot_general(w2t_ref[:], h1, (((1,), (0,)), ((), ())),
                                preferred_element_type=jnp.float32) + b2_ref[:]
    rgb = jax.nn.sigmoid(logit)
    argb_ref[:] = alpha * rgb


@functools.partial(jax.jit, static_argnames=())
def _tc_stage(density, k0_feat, viewdirs, W0, b0, W1, b1, W2, b2):
    grid = (S // _BLK,)
    log1m2d, argbT = pl.pallas_call(
        _tc_body,
        grid=grid,
        in_specs=[
            pl.BlockSpec((1, _BLK), lambda i: (0, i)),
            pl.BlockSpec((K0_DIM, _BLK), lambda i: (0, i)),
            pl.BlockSpec((3, _BLK), lambda i: (0, i)),
            pl.BlockSpec((RGBNET_WIDTH, 39), lambda i: (0, 0)),
            pl.BlockSpec((RGBNET_WIDTH, 1), lambda i: (0, 0)),
            pl.BlockSpec((RGBNET_WIDTH, RGBNET_WIDTH), lambda i: (0, 0)),
            pl.BlockSpec((RGBNET_WIDTH, 1), lambda i: (0, 0)),
            pl.BlockSpec((3, RGBNET_WIDTH), lambda i: (0, 0)),
            pl.BlockSpec((3, 1), lambda i: (0, 0)),
        ],
        out_specs=[
            pl.BlockSpec((1, _BLK), lambda i: (0, i)),
            pl.BlockSpec((3, _BLK), lambda i: (0, i)),
        ],
        out_shape=[
            jax.ShapeDtypeStruct((1, S), jnp.float32),
            jax.ShapeDtypeStruct((3, S), jnp.float32),
        ],
    )(density[None, :], k0_feat.T, viewdirs.T, W0[_W0_PERM].T, b0[:, None],
      W1.T, b1[:, None], W2.T, b2[:, None])
    return log1m2d.reshape(S), argbT


_NW = 16
_CHUNK = S // _NW
_NV = _CHUNK // 16
_SUB = 4096
_NSUB = _CHUNK // _SUB
_SUBV = _SUB // 16
_RPW = N_RAYS // _NW
_RG = _RPW // 16


def _sc_body(rid_hbm, lg_hbm, argb_hbm, out_hbm,
             rid_buf, ex_buf, argb_buf,
             off_loc, lastc_loc, acc_loc, off_tab,
             red, tot_v, fin_v, out_v, sc17, tot_stage,
             totals_sh, sh_big, fin_sh):
    cid = lax.axis_index("c")
    sid = lax.axis_index("s")

    @pl.when(cid == 0)
    def _core0():
        w = sid
        base = w * _CHUNK
        iota = lax.iota(jnp.int32, 16)
        zf = jnp.zeros((16,), jnp.float32)

        def _z(i, c):
            off_loc[pl.ds(16 * i, 16)] = zf
            lastc_loc[pl.ds(16 * i, 16)] = zf
            for ch in range(3):
                acc_loc[ch, pl.ds(16 * i, 16)] = zf
            return c
        lax.fori_loop(0, N_RAYS // 16, _z, 0)
        sc17[pl.ds(0, 16)] = jnp.full((16,), -1.0, jnp.float32)

        pltpu.sync_copy(rid_hbm.at[pl.ds(base, _CHUNK)],
                        rid_buf.at[pl.ds(16, _CHUNK)])

        @pl.when(w > 0)
        def _():
            pltpu.sync_copy(rid_hbm.at[pl.ds(base - 16, 16)],
                            rid_buf.at[pl.ds(0, 16)])

        @pl.when(w == 0)
        def _():
            rid_buf[pl.ds(0, 16)] = jnp.full((16,), -1, jnp.int32)

        @pl.when(w < _NW - 1)
        def _():
            pltpu.sync_copy(rid_hbm.at[pl.ds(base + _CHUNK, 16)],
                            rid_buf.at[pl.ds(16 + _CHUNK, 16)])

        @pl.when(w == _NW - 1)
        def _():
            rid_buf[pl.ds(16 + _CHUNK, 16)] = jnp.full((16,), -1, jnp.int32)

        def _p1sub(sub, carry):
            pltpu.sync_copy(lg_hbm.at[pl.ds(base + sub * _SUB, _SUB)],
                            ex_buf.at[pl.ds(sub * _SUB, _SUB)])

            def _p1v(j, c):
                o = sub * _SUB + 16 * j
                v = ex_buf[pl.ds(o, 16)]
                s = plsc.cumsum(v)
                ex_buf[pl.ds(o, 16)] = (c + s) - v
                return c + jnp.sum(v)
            return lax.fori_loop(0, _SUBV, _p1v, carry)
        total = lax.fori_loop(0, _NSUB, _p1sub, jnp.float32(0.0))
        ex_buf[pl.ds(_CHUNK, 16)] = zf + total

        tot_stage[pl.ds(0, 16)] = zf + total
        pltpu.sync_copy(tot_stage, totals_sh.at[w])
        plsc.subcore_barrier()
        pltpu.sync_copy(totals_sh, tot_v)
        tvals = plsc.load_gather(tot_v, [iota, jnp.zeros((16,), jnp.int32)])
        prefix = jnp.sum(jnp.where(iota < w, tvals, 0.0))

        def _p2(j, c):
            rv = rid_buf[pl.ds(16 + 16 * j, 16)]
            rp = plsc.load_gather(rid_buf, [15 + 16 * j + iota])
            rn = plsc.load_gather(rid_buf, [17 + 16 * j + iota])
            exg = ex_buf[pl.ds(16 * j, 16)] + prefix
            exn = plsc.load_gather(ex_buf, [16 * j + 1 + iota]) + prefix
            plsc.store_scatter(off_loc, [rv], exg, mask=rv != rp)
            plsc.store_scatter(lastc_loc, [rv], exn, mask=rv != rn)
            return c
        lax.fori_loop(0, _NV, _p2, 0)
        pltpu.sync_copy(off_loc, sh_big.at[w, 0])
        pltpu.sync_copy(lastc_loc, sh_big.at[w, 1])
        plsc.subcore_barrier()

        c0 = w * _RPW
        for t in range(2):
            for k in range(_NW):
                pltpu.sync_copy(sh_big.at[k, t, pl.ds(c0, _RPW)], red.at[k])
            for g in range(_RG):
                def _radd(k, a, g=g):
                    return a + red[k, pl.ds(16 * g, 16)]
                fin_v[t, pl.ds(16 * g, 16)] = lax.fori_loop(0, _NW, _radd, zf)
        pltpu.sync_copy(fin_v.at[0], fin_sh.at[0, pl.ds(c0, _RPW)])
        pltpu.sync_copy(fin_v.at[1], fin_sh.at[1, pl.ds(c0, _RPW)])
        plsc.subcore_barrier()
        pltpu.sync_copy(fin_sh.at[0], off_tab)

        def _p3sub(sub, carry):
            for ch in range(3):
                pltpu.sync_copy(
                    argb_hbm.at[pl.ds(ch * S + base + sub * _SUB, _SUB)],
                    argb_buf.at[pl.ds(ch * _SUB, _SUB)])

            def _p3v(j, cr):
                accs = [cr[0], cr[1], cr[2]]
                pbs = [cr[3], cr[4], cr[5]]
                jj = sub * _SUBV + j
                rv = rid_buf[pl.ds(16 + 16 * jj, 16)]
                rn = plsc.load_gather(rid_buf, [17 + 16 * jj + iota])
                exg = ex_buf[pl.ds(16 * jj, 16)] + prefix
                og = plsc.load_gather(off_tab, [rv])
                tw = jnp.exp(exg - og)
                lf = jnp.logical_or(
                    rv != rn,
                    jnp.logical_and(jj == _NV - 1, iota == 15))
                out = []
                for ch in range(3):
                    a = argb_buf[pl.ds(ch * _SUB + 16 * j, 16)]
                    v = tw * a
                    s = plsc.cumsum(v) + accs[ch]
                    wv = jnp.where(lf, s, -1.0)
                    plsc.store_scatter(sc17, [iota + 1], wv)
                    prevc = jnp.maximum(plsc.cummax(sc17[pl.ds(0, 16)]),
                                        pbs[ch])
                    plsc.store_scatter(
                        acc_loc, [jnp.full((16,), ch, jnp.int32), rv],
                        s - prevc, mask=lf)
                    out.append((accs[ch] + jnp.sum(v),
                                jnp.maximum(pbs[ch], jnp.max(wv))))
                return (out[0][0], out[1][0], out[2][0],
                        out[0][1], out[1][1], out[2][1])
            return lax.fori_loop(0, _SUBV, _p3v, carry)
        z6 = (jnp.float32(0.0),) * 6
        lax.fori_loop(0, _NSUB, _p3sub, z6)
        pltpu.sync_copy(acc_loc, sh_big.at[w])
        plsc.subcore_barrier()

        pltpu.sync_copy(fin_sh.at[0, pl.ds(c0, _RPW)], fin_v.at[0])
        pltpu.sync_copy(fin_sh.at[1, pl.ds(c0, _RPW)], fin_v.at[1])
        for g in range(_RG):
            offv = fin_v[0, pl.ds(16 * g, 16)]
            lcv = fin_v[1, pl.ds(16 * g, 16)]
            fin_v[0, pl.ds(16 * g, 16)] = jnp.exp(lcv - offv)
        for ch in range(3):
            for k in range(_NW):
                pltpu.sync_copy(sh_big.at[k, ch, pl.ds(c0, _RPW)], red.at[k])
            for g in range(_RG):
                def _r3(k, a, g=g):
                    return a + red[k, pl.ds(16 * g, 16)]
                av = lax.fori_loop(0, _NW, _r3, zf)
                rows = 16 * g + iota
                plsc.store_scatter(out_v, [3 * rows + ch],
                                   av + fin_v[0, pl.ds(16 * g, 16)])
        pltpu.sync_copy(out_v, out_hbm.at[pl.ds(c0 * 3, _RPW * 3)])


_sc_stage = pl.kernel(
    _sc_body,
    out_type=jax.ShapeDtypeStruct((N_RAYS * 3,), jnp.float32),
    mesh=plsc.VectorSubcoreMesh(core_axis_name="c", subcore_axis_name="s"),
    compiler_params=pltpu.CompilerParams(
        needs_layout_passes=False, use_tc_tiling_on_sc=False),
    scratch_types=[
        pltpu.VMEM((_CHUNK + 32,), jnp.int32),
        pltpu.VMEM((_CHUNK + 16,), jnp.float32),
        pltpu.VMEM((_SUB * 3,), jnp.float32),
        pltpu.VMEM((N_RAYS,), jnp.float32),
        pltpu.VMEM((N_RAYS,), jnp.float32),
        pltpu.VMEM((3, N_RAYS), jnp.float32),
        pltpu.VMEM((N_RAYS,), jnp.float32),
        pltpu.VMEM((_NW, _RPW), jnp.float32),
        pltpu.VMEM((16, 16), jnp.float32),
        pltpu.VMEM((2, _RPW), jnp.float32),
        pltpu.VMEM((_RPW * 3,), jnp.float32),
        pltpu.VMEM((32,), jnp.float32),
        pltpu.VMEM((16,), jnp.float32),
        pltpu.VMEM_SHARED((16, 16), jnp.float32),
        pltpu.VMEM_SHARED((_NW, 3, N_RAYS), jnp.float32),
        pltpu.VMEM_SHARED((2, N_RAYS), jnp.float32),
    ],
)


def kernel(density, k0_feat, viewdirs, ray_id, W0, b0, W1, b1, W2, b2):
    log1m, argbT = _tc_stage(density, k0_feat, viewdirs, W0, b0, W1, b1, W2, b2)
    flat = _sc_stage(ray_id, log1m, argbT.reshape(-1))
    return flat.reshape(N_RAYS, 3)


# speedup vs baseline: 12.1345x; 1.0988x over previous
import functools

import jax
import jax.numpy as jnp
import numpy as np
from jax import lax
from jax.experimental import pallas as pl
from jax.experimental.pallas import tpu as pltpu
from jax.experimental.pallas import tpu_sc as plsc

N_RAYS = 4096
S = 524288
K0_DIM = 12
VIEWBASE_PE = 4
RGBNET_WIDTH = 128
ALPHA_INIT = 1e-6
_ACT_SHIFT = float(np.log(1.0 / (1.0 - ALPHA_INIT) - 1.0))

_W0_PERM = np.arange(39)
for _b in (15, 27):
    for _f in range(VIEWBASE_PE):
        for _d in range(3):
            _W0_PERM[_b + 3 * _f + _d] = _b + _d * 4 + _f

_BLK = 8192


def _tc_body(dens_ref, k0t_ref, vdt_ref, w0t_ref, b0_ref, w1t_ref, b1_ref,
             w2t_ref, b2_ref, log1m_ref, argb_ref):
    x = dens_ref[:]
    e = jnp.exp(x + _ACT_SHIFT)
    t = jax.lax.rsqrt(1.0 + e)
    alpha = 1.0 - t
    log1m_ref[:] = jnp.log(jnp.clip(t, 1e-10, 1.0))

    vd = vdt_ref[:]
    norm = jnp.sqrt(jnp.sum(vd * vd, axis=0, keepdims=True)) + 1e-8
    vdn = vd / norm

    s = jnp.sin(vdn)
    c = jnp.cos(vdn)
    sc = [(s, c)]
    for _ in range(VIEWBASE_PE - 1):
        s, c = 2.0 * s * c, 1.0 - 2.0 * s * s
        sc.append((s, c))

    w0t = w0t_ref[:]
    h0 = (jax.lax.dot_general(w0t[:, 0:12], k0t_ref[:],
                              (((1,), (0,)), ((), ())),
                              preferred_element_type=jnp.float32)
          + jax.lax.dot_general(w0t[:, 12:15], vdn,
                                (((1,), (0,)), ((), ())),
                                preferred_element_type=jnp.float32)
          + b0_ref[:])
    for f in range(VIEWBASE_PE):
        h0 = h0 + jax.lax.dot_general(w0t[:, 15 + 3 * f:18 + 3 * f],
                                      sc[f][0], (((1,), (0,)), ((), ())),
                                      preferred_element_type=jnp.float32)
        h0 = h0 + jax.lax.dot_general(w0t[:, 27 + 3 * f:30 + 3 * f],
                                      sc[f][1], (((1,), (0,)), ((), ())),
                                      preferred_element_type=jnp.float32)
    h0 = jnp.maximum(h0, 0.0)
    h1 = jnp.maximum(
        jax.lax.dot_general(w1t_ref[:], h0, (((1,), (0,)), ((), ())),
                            preferred_element_type=jnp.float32) + b1_ref[:],
        0.0)
    logit = jax.lax.dot_general(w2t_ref[:], h1, (((1,), (0,)), ((), ())),
                                preferred_element_type=jnp.float32) + b2_ref[:]
    rgb = jax.nn.sigmoid(logit)
    argb_ref[:] = alpha * rgb


@functools.partial(jax.jit, static_argnames=())
def _tc_stage(density, k0_feat, viewdirs, W0, b0, W1, b1, W2, b2):
    grid = (S // _BLK,)
    log1m2d, argbT = pl.pallas_call(
        _tc_body,
        grid=grid,
        in_specs=[
            pl.BlockSpec((1, _BLK), lambda i: (0, i)),
            pl.BlockSpec((K0_DIM, _BLK), lambda i: (0, i)),
            pl.BlockSpec((3, _BLK), lambda i: (0, i)),
            pl.BlockSpec((RGBNET_WIDTH, 39), lambda i: (0, 0)),
            pl.BlockSpec((RGBNET_WIDTH, 1), lambda i: (0, 0)),
            pl.BlockSpec((RGBNET_WIDTH, RGBNET_WIDTH), lambda i: (0, 0)),
            pl.BlockSpec((RGBNET_WIDTH, 1), lambda i: (0, 0)),
            pl.BlockSpec((3, RGBNET_WIDTH), lambda i: (0, 0)),
            pl.BlockSpec((3, 1), lambda i: (0, 0)),
        ],
        out_specs=[
            pl.BlockSpec((1, _BLK), lambda i: (0, i)),
            pl.BlockSpec((3, _BLK), lambda i: (0, i)),
        ],
        out_shape=[
            jax.ShapeDtypeStruct((1, S), jnp.float32),
            jax.ShapeDtypeStruct((3, S), jnp.float32),
        ],
    )(density[None, :], k0_feat.T, viewdirs.T, W0[_W0_PERM].T, b0[:, None],
      W1.T, b1[:, None], W2.T, b2[:, None])
    return log1m2d.reshape(S), argbT


_NW = 16
_CHUNK = S // _NW
_NV = _CHUNK // 16
_SUB = 4096
_NSUB = _CHUNK // _SUB
_SUBV = _SUB // 16
_RPW = N_RAYS // _NW
_RG = _RPW // 16


def _sc_body(rid_hbm, lg_hbm, argb_hbm, out_hbm,
             rid_buf, ex_buf, argb_buf,
             off_loc, lastc_loc, acc_loc, off_tab,
             red, tot_v, fin_v, out_v, sc17, tot_stage,
             totals_sh, sh_big, fin_sh):
    cid = lax.axis_index("c")
    sid = lax.axis_index("s")

    @pl.when(cid == 0)
    def _core0():
        w = sid
        base = w * _CHUNK
        iota = lax.iota(jnp.int32, 16)
        zf = jnp.zeros((16,), jnp.float32)

        def _z(i, c):
            off_loc[pl.ds(16 * i, 16)] = zf
            lastc_loc[pl.ds(16 * i, 16)] = zf
            for ch in range(3):
                acc_loc[ch, pl.ds(16 * i, 16)] = zf
            return c
        lax.fori_loop(0, N_RAYS // 16, _z, 0)
        sc17[pl.ds(0, 16)] = jnp.full((16,), -1.0, jnp.float32)

        pltpu.sync_copy(rid_hbm.at[pl.ds(base, _CHUNK)],
                        rid_buf.at[pl.ds(16, _CHUNK)])

        @pl.when(w > 0)
        def _():
            pltpu.sync_copy(rid_hbm.at[pl.ds(base - 16, 16)],
                            rid_buf.at[pl.ds(0, 16)])

        @pl.when(w == 0)
        def _():
            rid_buf[pl.ds(0, 16)] = jnp.full((16,), -1, jnp.int32)

        @pl.when(w < _NW - 1)
        def _():
            pltpu.sync_copy(rid_hbm.at[pl.ds(base + _CHUNK, 16)],
                            rid_buf.at[pl.ds(16 + _CHUNK, 16)])

        @pl.when(w == _NW - 1)
        def _():
            rid_buf[pl.ds(16 + _CHUNK, 16)] = jnp.full((16,), -1, jnp.int32)

        def _p1sub(sub, carry):
            pltpu.sync_copy(lg_hbm.at[pl.ds(base + sub * _SUB, _SUB)],
                            ex_buf.at[pl.ds(sub * _SUB, _SUB)])

            def _p1v(j, c):
                o = sub * _SUB + 16 * j
                v = ex_buf[pl.ds(o, 16)]
                s = plsc.cumsum(v)
                ex_buf[pl.ds(o, 16)] = (c + s) - v
                return c + jnp.sum(v)
            return lax.fori_loop(0, _SUBV, _p1v, carry)
        total = lax.fori_loop(0, _NSUB, _p1sub, jnp.float32(0.0))
        ex_buf[pl.ds(_CHUNK, 16)] = zf + total

        tot_stage[pl.ds(0, 16)] = zf + total
        pltpu.sync_copy(tot_stage, totals_sh.at[w])
        plsc.subcore_barrier()
        pltpu.sync_copy(totals_sh, tot_v)
        tvals = plsc.load_gather(tot_v, [iota, jnp.zeros((16,), jnp.int32)])
        prefix = jnp.sum(jnp.where(iota < w, tvals, 0.0))

        def _p2(j, c):
            rv = rid_buf[pl.ds(16 + 16 * j, 16)]
            rp = plsc.load_gather(rid_buf, [15 + 16 * j + iota])
            rn = plsc.load_gather(rid_buf, [17 + 16 * j + iota])
            exg = ex_buf[pl.ds(16 * j, 16)] + prefix
            exn = plsc.load_gather(ex_buf, [16 * j + 1 + iota]) + prefix
            plsc.store_scatter(off_loc, [rv], exg, mask=rv != rp)
            plsc.store_scatter(lastc_loc, [rv], exn, mask=rv != rn)
            return c
        lax.fori_loop(0, _NV, _p2, 0)
        pltpu.sync_copy(off_loc, sh_big.at[w, 0])
        pltpu.sync_copy(lastc_loc, sh_big.at[w, 1])
        plsc.subcore_barrier()

        c0 = w * _RPW
        for t in range(2):
            for k in range(_NW):
                pltpu.sync_copy(sh_big.at[k, t, pl.ds(c0, _RPW)], red.at[k])
            for g in range(_RG):
                def _radd(k, a, g=g):
                    return a + red[k, pl.ds(16 * g, 16)]
                fin_v[t, pl.ds(16 * g, 16)] = lax.fori_loop(0, _NW, _radd, zf)
        pltpu.sync_copy(fin_v.at[0], fin_sh.at[0, pl.ds(c0, _RPW)])
        pltpu.sync_copy(fin_v.at[1], fin_sh.at[1, pl.ds(c0, _RPW)])
        plsc.subcore_barrier()
        pltpu.sync_copy(fin_sh.at[0], off_tab)

        def _p3sub(sub, carry):
            for ch in range(3):
                pltpu.sync_copy(
                    argb_hbm.at[pl.ds(ch * S + base + sub * _SUB, _SUB)],
                    argb_buf.at[pl.ds(ch * _SUB, _SUB)])

            def _p3v(j, cr):
                accs = [cr[0], cr[1], cr[2]]
                pbs = [cr[3], cr[4], cr[5]]
                jj = sub * _SUBV + j
                rv = rid_buf[pl.ds(16 + 16 * jj, 16)]
                rn = plsc.load_gather(rid_buf, [17 + 16 * jj + iota])
                exg = ex_buf[pl.ds(16 * jj, 16)] + prefix
                og = plsc.load_gather(off_tab, [rv])
                tw = jnp.exp(exg - og)
                lf = jnp.logical_or(
                    rv != rn,
                    jnp.logical_and(jj == _NV - 1, iota == 15))
                out = []
                for ch in range(3):
                    a = argb_buf[pl.ds(ch * _SUB + 16 * j, 16)]
                    v = tw * a
                    s = plsc.cumsum(v) + accs[ch]
                    wv = jnp.where(lf, s, -1.0)
                    plsc.store_scatter(sc17, [iota + 1], wv)
                    prevc = jnp.maximum(plsc.cummax(sc17[pl.ds(0, 16)]),
                                        pbs[ch])
                    plsc.store_scatter(
                        acc_loc, [jnp.full((16,), ch, jnp.int32), rv],
                        s - prevc, mask=lf)
                    out.append((accs[ch] + jnp.sum(v),
                                jnp.maximum(pbs[ch], jnp.max(wv))))
                return (out[0][0], out[1][0], out[2][0],
                        out[0][1], out[1][1], out[2][1])
            return lax.fori_loop(0, _SUBV, _p3v, carry)
        z6 = (jnp.float32(0.0),) * 6
        lax.fori_loop(0, _NSUB, _p3sub, z6)
        pltpu.sync_copy(acc_loc, sh_big.at[w])
        plsc.subcore_barrier()

        pltpu.sync_copy(fin_sh.at[0, pl.ds(c0, _RPW)], fin_v.at[0])
        pltpu.sync_copy(fin_sh.at[1, pl.ds(c0, _RPW)], fin_v.at[1])
        for g in range(_RG):
            offv = fin_v[0, pl.ds(16 * g, 16)]
            lcv = fin_v[1, pl.ds(16 * g, 16)]
            fin_v[0, pl.ds(16 * g, 16)] = jnp.exp(lcv - offv)
        for ch in range(3):
            for k in range(_NW):
                pltpu.sync_copy(sh_big.at[k, ch, pl.ds(c0, _RPW)], red.at[k])
            for g in range(_RG):
                def _r3(k, a, g=g):
                    return a + red[k, pl.ds(16 * g, 16)]
                av = lax.fori_loop(0, _NW, _r3, zf)
                rows = 16 * g + iota
                plsc.store_scatter(out_v, [3 * rows + ch],
                                   av + fin_v[0, pl.ds(16 * g, 16)])
        pltpu.sync_copy(out_v, out_hbm.at[pl.ds(c0 * 3, _RPW * 3)])


_sc_stage = pl.kernel(
    _sc_body,
    out_type=jax.ShapeDtypeStruct((N_RAYS * 3,), jnp.float32),
    mesh=plsc.VectorSubcoreMesh(core_axis_name="c", subcore_axis_name="s"),
    compiler_params=pltpu.CompilerParams(
        needs_layout_passes=False, use_tc_tiling_on_sc=False),
    scratch_types=[
        pltpu.VMEM((_CHUNK + 32,), jnp.int32),
        pltpu.VMEM((_CHUNK + 16,), jnp.float32),
        pltpu.VMEM((_SUB * 3,), jnp.float32),
        pltpu.VMEM((N_RAYS,), jnp.float32),
        pltpu.VMEM((N_RAYS,), jnp.float32),
        pltpu.VMEM((3, N_RAYS), jnp.float32),
        pltpu.VMEM((N_RAYS,), jnp.float32),
        pltpu.VMEM((_NW, _RPW), jnp.float32),
        pltpu.VMEM((16, 16), jnp.float32),
        pltpu.VMEM((2, _RPW), jnp.float32),
        pltpu.VMEM((_RPW * 3,), jnp.float32),
        pltpu.VMEM((32,), jnp.float32),
        pltpu.VMEM((16,), jnp.float32),
        pltpu.VMEM_SHARED((16, 16), jnp.float32),
        pltpu.VMEM_SHARED((_NW, 3, N_RAYS), jnp.float32),
        pltpu.VMEM_SHARED((2, N_RAYS), jnp.float32),
    ],
)


def kernel(density, k0_feat, viewdirs, ray_id, W0, b0, W1, b1, W2, b2):
    log1m, argbT = _tc_stage(density, k0_feat, viewdirs, W0, b0, W1, b1, W2, b2)
    flat = _sc_stage(ray_id, log1m, argbT.reshape(-1))
    return flat.reshape(N_RAYS, 3)


# speedup vs baseline: 12.1601x; 1.0021x over previous
import functools

import jax
import jax.numpy as jnp
import numpy as np
from jax import lax
from jax.experimental import pallas as pl
from jax.experimental.pallas import tpu as pltpu
from jax.experimental.pallas import tpu_sc as plsc

N_RAYS = 4096
S = 524288
K0_DIM = 12
VIEWBASE_PE = 4
RGBNET_WIDTH = 128
ALPHA_INIT = 1e-6
_ACT_SHIFT = float(np.log(1.0 / (1.0 - ALPHA_INIT) - 1.0))

_W0_PERM = np.arange(39)
for _b in (15, 27):
    for _f in range(VIEWBASE_PE):
        for _d in range(3):
            _W0_PERM[_b + 3 * _f + _d] = _b + _d * 4 + _f

_BLK = 8192


def _tc_body(dens_ref, k0t_ref, vdt_ref, w0t_ref, b0_ref, w1t_ref, b1_ref,
             w2t_ref, b2_ref, log1m_ref, argb_ref):
    x = dens_ref[:]
    e = jnp.exp(x + _ACT_SHIFT)
    t = jax.lax.rsqrt(1.0 + e)
    alpha = 1.0 - t
    log1m_ref[:] = jnp.log(jnp.clip(t, 1e-10, 1.0))

    vd = vdt_ref[:]
    norm = jnp.sqrt(jnp.sum(vd * vd, axis=0, keepdims=True)) + 1e-8
    vdn = vd / norm

    s = jnp.sin(vdn)
    c = jnp.cos(vdn)
    sc = [(s, c)]
    for _ in range(VIEWBASE_PE - 1):
        s, c = 2.0 * s * c, 1.0 - 2.0 * s * s
        sc.append((s, c))

    w0t = w0t_ref[:]
    h0 = (jax.lax.dot_general(w0t[:, 0:12], k0t_ref[:],
                              (((1,), (0,)), ((), ())),
                              preferred_element_type=jnp.float32)
          + jax.lax.dot_general(w0t[:, 12:15], vdn,
                                (((1,), (0,)), ((), ())),
                                preferred_element_type=jnp.float32)
          + b0_ref[:])
    for f in range(VIEWBASE_PE):
        h0 = h0 + jax.lax.dot_general(w0t[:, 15 + 3 * f:18 + 3 * f],
                                      sc[f][0], (((1,), (0,)), ((), ())),
                                      preferred_element_type=jnp.float32)
        h0 = h0 + jax.lax.dot_general(w0t[:, 27 + 3 * f:30 + 3 * f],
                                      sc[f][1], (((1,), (0,)), ((), ())),
                                      preferred_element_type=jnp.float32)
    h0 = jnp.maximum(h0, 0.0)
    h1 = jnp.maximum(
        jax.lax.dot_general(w1t_ref[:], h0, (((1,), (0,)), ((), ())),
                            preferred_element_type=jnp.float32) + b1_ref[:],
        0.0)
    logit = jax.lax.dot_general(w2t_ref[:], h1, (((1,), (0,)), ((), ())),
                                preferred_element_type=jnp.float32) + b2_ref[:]
    rgb = jax.nn.sigmoid(logit)
    argb_ref[:] = alpha * rgb


@functools.partial(jax.jit, static_argnames=())
def _tc_stage(density, k0_feat, viewdirs, W0, b0, W1, b1, W2, b2):
    grid = (S // _BLK,)
    log1m2d, argbT = pl.pallas_call(
        _tc_body,
        grid=grid,
        in_specs=[
            pl.BlockSpec((1, _BLK), lambda i: (0, i)),
            pl.BlockSpec((K0_DIM, _BLK), lambda i: (0, i)),
            pl.BlockSpec((3, _BLK), lambda i: (0, i)),
            pl.BlockSpec((RGBNET_WIDTH, 39), lambda i: (0, 0)),
            pl.BlockSpec((RGBNET_WIDTH, 1), lambda i: (0, 0)),
            pl.BlockSpec((RGBNET_WIDTH, RGBNET_WIDTH), lambda i: (0, 0)),
            pl.BlockSpec((RGBNET_WIDTH, 1), lambda i: (0, 0)),
            pl.BlockSpec((3, RGBNET_WIDTH), lambda i: (0, 0)),
            pl.BlockSpec((3, 1), lambda i: (0, 0)),
        ],
        out_specs=[
            pl.BlockSpec((1, _BLK), lambda i: (0, i)),
            pl.BlockSpec((3, _BLK), lambda i: (0, i)),
        ],
        out_shape=[
            jax.ShapeDtypeStruct((1, S), jnp.float32),
            jax.ShapeDtypeStruct((3, S), jnp.float32),
        ],
    )(density[None, :], k0_feat.T, viewdirs.T, W0[_W0_PERM].T, b0[:, None],
      W1.T, b1[:, None], W2.T, b2[:, None])
    return log1m2d.reshape(S), argbT


_NW = 16
_CHUNK = S // _NW
_NV = _CHUNK // 16
_SUB = 4096
_NSUB = _CHUNK // _SUB
_SUBV = _SUB // 16
_RPW = N_RAYS // _NW
_RG = _RPW // 16


def _sc_body(rid_hbm, lg_hbm, argb_hbm, out_hbm,
             rid_buf, ex_buf, argb_buf,
             off_loc, lastc_loc, acc_loc, off_tab,
             red, tot_v, fin_v, out_v, sc17, tot_stage,
             totals_sh, sh_big, fin_sh):
    cid = lax.axis_index("c")
    sid = lax.axis_index("s")

    @pl.when(cid == 0)
    def _core0():
        w = sid
        base = w * _CHUNK
        iota = lax.iota(jnp.int32, 16)
        zf = jnp.zeros((16,), jnp.float32)

        def _z(i, c):
            off_loc[pl.ds(16 * i, 16)] = zf
            lastc_loc[pl.ds(16 * i, 16)] = zf
            for ch in range(3):
                acc_loc[ch, pl.ds(16 * i, 16)] = zf
            return c
        lax.fori_loop(0, N_RAYS // 16, _z, 0)
        sc17[pl.ds(0, 16)] = jnp.full((16,), -1.0, jnp.float32)

        pltpu.sync_copy(rid_hbm.at[pl.ds(base, _CHUNK)],
                        rid_buf.at[pl.ds(16, _CHUNK)])

        @pl.when(w > 0)
        def _():
            pltpu.sync_copy(rid_hbm.at[pl.ds(base - 16, 16)],
                            rid_buf.at[pl.ds(0, 16)])

        @pl.when(w == 0)
        def _():
            rid_buf[pl.ds(0, 16)] = jnp.full((16,), -1, jnp.int32)

        @pl.when(w < _NW - 1)
        def _():
            pltpu.sync_copy(rid_hbm.at[pl.ds(base + _CHUNK, 16)],
                            rid_buf.at[pl.ds(16 + _CHUNK, 16)])

        @pl.when(w == _NW - 1)
        def _():
            rid_buf[pl.ds(16 + _CHUNK, 16)] = jnp.full((16,), -1, jnp.int32)

        def _p1sub(sub, carry):
            pltpu.sync_copy(lg_hbm.at[pl.ds(base + sub * _SUB, _SUB)],
                            ex_buf.at[pl.ds(sub * _SUB, _SUB)])

            def _p1v(j, c):
                o = sub * _SUB + 16 * j
                v = ex_buf[pl.ds(o, 16)]
                s = plsc.cumsum(v)
                ex_buf[pl.ds(o, 16)] = (c + s) - v
                return c + jnp.sum(v)
            return lax.fori_loop(0, _SUBV, _p1v, carry)
        total = lax.fori_loop(0, _NSUB, _p1sub, jnp.float32(0.0))
        ex_buf[pl.ds(_CHUNK, 16)] = zf + total

        tot_stage[pl.ds(0, 16)] = zf + total
        pltpu.sync_copy(tot_stage, totals_sh.at[w])
        plsc.subcore_barrier()
        pltpu.sync_copy(totals_sh, tot_v)
        tvals = plsc.load_gather(tot_v, [iota, jnp.zeros((16,), jnp.int32)])
        prefix = jnp.sum(jnp.where(iota < w, tvals, 0.0))

        def _p2(j, c):
            rv = rid_buf[pl.ds(16 + 16 * j, 16)]
            rp = plsc.load_gather(rid_buf, [15 + 16 * j + iota])
            rn = plsc.load_gather(rid_buf, [17 + 16 * j + iota])
            bf = rv != rp
            lf = rv != rn

            def _bnd(_):
                exg = ex_buf[pl.ds(16 * j, 16)] + prefix
                exn = plsc.load_gather(ex_buf, [16 * j + 1 + iota]) + prefix
                plsc.store_scatter(off_loc, [rv], exg, mask=bf)
                plsc.store_scatter(lastc_loc, [rv], exn, mask=lf)
                return 0

            return lax.cond(jnp.any(jnp.logical_or(bf, lf)),
                            _bnd, lambda _: 0, 0)
        lax.fori_loop(0, _NV, _p2, 0)
        pltpu.sync_copy(off_loc, sh_big.at[w, 0])
        pltpu.sync_copy(lastc_loc, sh_big.at[w, 1])
        plsc.subcore_barrier()

        c0 = w * _RPW
        for t in range(2):
            for k in range(_NW):
                pltpu.sync_copy(sh_big.at[k, t, pl.ds(c0, _RPW)], red.at[k])
            for g in range(_RG):
                def _radd(k, a, g=g):
                    return a + red[k, pl.ds(16 * g, 16)]
                fin_v[t, pl.ds(16 * g, 16)] = lax.fori_loop(0, _NW, _radd, zf)
        pltpu.sync_copy(fin_v.at[0], fin_sh.at[0, pl.ds(c0, _RPW)])
        pltpu.sync_copy(fin_v.at[1], fin_sh.at[1, pl.ds(c0, _RPW)])
        plsc.subcore_barrier()
        pltpu.sync_copy(fin_sh.at[0], off_tab)

        def _p3sub(sub, carry):
            for ch in range(3):
                pltpu.sync_copy(
                    argb_hbm.at[pl.ds(ch * S + base + sub * _SUB, _SUB)],
                    argb_buf.at[pl.ds(ch * _SUB, _SUB)])

            def _p3v(j, cr):
                accs = [cr[0], cr[1], cr[2]]
                pbs = [cr[3], cr[4], cr[5]]
                jj = sub * _SUBV + j
                rv = rid_buf[pl.ds(16 + 16 * jj, 16)]
                rn = plsc.load_gather(rid_buf, [17 + 16 * jj + iota])
                exg = ex_buf[pl.ds(16 * jj, 16)] + prefix
                og = plsc.load_gather(off_tab, [rv])
                tw = jnp.exp(exg - og)
                lf = jnp.logical_or(
                    rv != rn,
                    jnp.logical_and(jj == _NV - 1, iota == 15))
                vs = [tw * argb_buf[pl.ds(ch * _SUB + 16 * j, 16)]
                      for ch in range(3)]

                def _slow(vs, accs, pbs):
                    out = []
                    for ch in range(3):
                        s = plsc.cumsum(vs[ch]) + accs[ch]
                        wv = jnp.where(lf, s, -1.0)
                        plsc.store_scatter(sc17, [iota + 1], wv)
                        prevc = jnp.maximum(plsc.cummax(sc17[pl.ds(0, 16)]),
                                            pbs[ch])
                        plsc.store_scatter(
                            acc_loc, [jnp.full((16,), ch, jnp.int32), rv],
                            s - prevc, mask=lf)
                        out.append((accs[ch] + jnp.sum(vs[ch]),
                                    jnp.maximum(pbs[ch], jnp.max(wv))))
                    return (out[0][0], out[1][0], out[2][0],
                            out[0][1], out[1][1], out[2][1])

                def _fast(vs, accs, pbs):
                    return (accs[0] + jnp.sum(vs[0]),
                            accs[1] + jnp.sum(vs[1]),
                            accs[2] + jnp.sum(vs[2]),
                            pbs[0], pbs[1], pbs[2])

                return lax.cond(jnp.any(lf), _slow, _fast, vs, accs, pbs)
            return lax.fori_loop(0, _SUBV, _p3v, carry)
        z6 = (jnp.float32(0.0),) * 6
        lax.fori_loop(0, _NSUB, _p3sub, z6)
        pltpu.sync_copy(acc_loc, sh_big.at[w])
        plsc.subcore_barrier()

        pltpu.sync_copy(fin_sh.at[0, pl.ds(c0, _RPW)], fin_v.at[0])
        pltpu.sync_copy(fin_sh.at[1, pl.ds(c0, _RPW)], fin_v.at[1])
        for g in range(_RG):
            offv = fin_v[0, pl.ds(16 * g, 16)]
            lcv = fin_v[1, pl.ds(16 * g, 16)]
            fin_v[0, pl.ds(16 * g, 16)] = jnp.exp(lcv - offv)
        for ch in range(3):
            for k in range(_NW):
                pltpu.sync_copy(sh_big.at[k, ch, pl.ds(c0, _RPW)], red.at[k])
            for g in range(_RG):
                def _r3(k, a, g=g):
                    return a + red[k, pl.ds(16 * g, 16)]
                av = lax.fori_loop(0, _NW, _r3, zf)
                rows = 16 * g + iota
                plsc.store_scatter(out_v, [3 * rows + ch],
                                   av + fin_v[0, pl.ds(16 * g, 16)])
        pltpu.sync_copy(out_v, out_hbm.at[pl.ds(c0 * 3, _RPW * 3)])


_sc_stage = pl.kernel(
    _sc_body,
    out_type=jax.ShapeDtypeStruct((N_RAYS * 3,), jnp.float32),
    mesh=plsc.VectorSubcoreMesh(core_axis_name="c", subcore_axis_name="s"),
    compiler_params=pltpu.CompilerParams(
        needs_layout_passes=False, use_tc_tiling_on_sc=False),
    scratch_types=[
        pltpu.VMEM((_CHUNK + 32,), jnp.int32),
        pltpu.VMEM((_CHUNK + 16,), jnp.float32),
        pltpu.VMEM((_SUB * 3,), jnp.float32),
        pltpu.VMEM((N_RAYS,), jnp.float32),
        pltpu.VMEM((N_RAYS,), jnp.float32),
        pltpu.VMEM((3, N_RAYS), jnp.float32),
        pltpu.VMEM((N_RAYS,), jnp.float32),
        pltpu.VMEM((_NW, _RPW), jnp.float32),
        pltpu.VMEM((16, 16), jnp.float32),
        pltpu.VMEM((2, _RPW), jnp.float32),
        pltpu.VMEM((_RPW * 3,), jnp.float32),
        pltpu.VMEM((32,), jnp.float32),
        pltpu.VMEM((16,), jnp.float32),
        pltpu.VMEM_SHARED((16, 16), jnp.float32),
        pltpu.VMEM_SHARED((_NW, 3, N_RAYS), jnp.float32),
        pltpu.VMEM_SHARED((2, N_RAYS), jnp.float32),
    ],
)


def kernel(density, k0_feat, viewdirs, ray_id, W0, b0, W1, b1, W2, b2):
    log1m, argbT = _tc_stage(density, k0_feat, viewdirs, W0, b0, W1, b1, W2, b2)
    flat = _sc_stage(ray_id, log1m, argbT.reshape(-1))
    return flat.reshape(N_RAYS, 3)
